# baseline (speedup 1.0000x reference)
import jax
import jax.numpy as jnp
from jax import lax
from jax.experimental import pallas as pl
from jax.experimental.pallas import tpu as pltpu
from jax.experimental.pallas import tpu_sc as plsc

N = 10000
E = 320000
F_IN = 128
F_OUT = 128
HALF = F_OUT // 2
NS = 16
NC = 2
ROWS_PER_TILE = 640
ROWS_LAST = N - 15 * ROWS_PER_TILE
EDGES_PER_TILE = E // NS
CHUNK = 80
NCHUNK = EDGES_PER_TILE // CHUNK
SLAB = 10
NSLAB = NCHUNK // SLAB


def _leaky(v):
    return jnp.where(v >= 0.0, v, 0.2 * v)


def _tc_pre_body(x_ref, w_ref, a_ref, feat_ref, s_ref, n_ref, m_ref):
    f = jnp.dot(x_ref[...], w_ref[...],
                preferred_element_type=jnp.float32,
                precision=lax.Precision.HIGHEST)
    feat_ref[...] = f
    attn = jnp.dot(f, a_ref[...],
                   preferred_element_type=jnp.float32,
                   precision=lax.Precision.HIGHEST)
    s_ref[...] = attn[:, 0:1]
    n_ref[...] = attn[:, 1:2]
    m_ref[...] = jnp.broadcast_to(jnp.max(attn[:, 1]), (1, 16))


_tc_pre = pl.pallas_call(
    _tc_pre_body,
    out_shape=[
        jax.ShapeDtypeStruct((N, F_OUT), jnp.float32),
        jax.ShapeDtypeStruct((N, 1), jnp.float32),
        jax.ShapeDtypeStruct((N, 1), jnp.float32),
        jax.ShapeDtypeStruct((1, 16), jnp.float32),
    ],
)


def _sc_edge_body(feat_hbm, s_hbm, n_hbm, m_hbm, row_hbm, col_hbm, adj_hbm,
                  b_hbm, out_hbm,
                  table, accS, denS,
                  s_v, n_v, m_v, b_v, d80, row_b, col_b, adj_b,
                  p_s, rows_a, rows_b, z1_v,
                  g_sem_a, g_sem_b, w_sem_a, w_sem_b):
    c = lax.axis_index("c")
    t = lax.axis_index("s")
    r0 = pl.multiple_of(t * ROWS_PER_TILE, 8)
    col0 = pl.multiple_of(c * HALF, 16)
    n_stage = ROWS_PER_TILE // CHUNK
    n_my = jnp.where(t < 15, n_stage, ROWS_LAST // CHUNK)

    z16 = jnp.zeros((16,), jnp.float32)
    for i in range(CHUNK):
        for k in range(HALF // 16):
            rows_a[i, pl.ds(k * 16, 16)] = z16

    def zero_blk(k, carry):
        pltpu.sync_copy(rows_a, accS.at[pl.ds(r0 + k * CHUNK, CHUNK)])
        return carry

    lax.fori_loop(0, n_my, zero_blk, 0)

    def stage_blk(k, carry):
        sl = pl.ds(r0 + k * CHUNK, CHUNK)
        pltpu.sync_copy(feat_hbm.at[sl, pl.ds(col0, HALF)], rows_b)
        pltpu.sync_copy(rows_b, table.at[sl])
        return carry

    lax.fori_loop(0, n_my, stage_blk, 0)

    @pl.when(t < 5)
    def _():
        for i in range(2000 // 16):
            z1_v[pl.ds(i * 16, 16)] = z16
        pltpu.sync_copy(z1_v, denS.at[pl.ds(t * 2000, 2000)])

    pltpu.sync_copy(s_hbm, s_v)
    pltpu.sync_copy(n_hbm, n_v)
    pltpu.sync_copy(m_hbm, m_v)
    pltpu.sync_copy(b_hbm, b_v)
    plsc.subcore_barrier()

    m = m_v[0, pl.ds(0, 16)][0]
    z16i = jnp.zeros((16,), jnp.int32)
    c0 = t * NCHUNK

    def scale(ci, rows_ref):
        for j in range(CHUNK // 16):
            p16 = p_s[ci, pl.ds(j * 16, 16)]
            for u in range(16):
                pe = p16[u]
                ei = j * 16 + u
                for k2 in range(HALF // 16):
                    sl2 = pl.ds(k2 * 16, 16)
                    rows_ref[ei, sl2] = rows_ref[ei, sl2] * pe

    def scatter(ci, rows_ref, sem):
        pltpu.async_copy(rows_ref, accS.at[row_b.at[ci]], sem, add=True)
        pltpu.async_copy(p_s.at[ci], denS.at[row_b.at[ci]], sem, add=True)

    def drain(rows_ref, sem):
        pltpu.make_async_copy(rows_ref, accS.at[row_b.at[0]], sem).wait()
        pltpu.make_async_copy(p_s.at[0], denS.at[row_b.at[0]], sem).wait()

    PAIRS = SLAB // 2

    def slab(sb, carry):
        @pl.when(sb > 0)
        def _():
            drain(rows_a, w_sem_a)
            drain(rows_b, w_sem_b)

        sl0 = pl.multiple_of(c0 + sb * SLAB, 2)
        pltpu.sync_copy(row_hbm.at[pl.ds(sl0, SLAB)], row_b)
        pltpu.sync_copy(col_hbm.at[pl.ds(sl0, SLAB)], col_b)
        pltpu.sync_copy(adj_hbm.at[pl.ds(sl0, SLAB)], adj_b)
        pltpu.async_copy(table.at[col_b.at[0]], rows_a, g_sem_a)

        for ci in range(SLAB):
            for j in range(CHUNK // 16):
                sl = pl.ds(j * 16, 16)
                a_s = plsc.load_gather(s_v, [row_b[ci, sl]])
                a_n = plsc.load_gather(n_v, [col_b[ci, sl]])
                e = _leaky(a_s + a_n) * adj_b[ci, sl]
                p_s[ci, sl] = jnp.exp(e - _leaky(a_s + m))

        def pair(i, carry2):
            ca = 2 * i
            cb = 2 * i + 1
            pltpu.make_async_copy(table.at[col_b.at[ca]], rows_a,
                                  g_sem_a).wait()

            @pl.when(i > 0)
            def _():
                drain(rows_b, w_sem_b)

            pltpu.async_copy(table.at[col_b.at[cb]], rows_b, g_sem_b)
            scale(ca, rows_a)
            scatter(ca, rows_a, w_sem_a)
            pltpu.make_async_copy(table.at[col_b.at[cb]], rows_b,
                                  g_sem_b).wait()

            @pl.when(i < PAIRS - 1)
            def _():
                drain(rows_a, w_sem_a)
                pltpu.async_copy(table.at[col_b.at[cb + 1]], rows_a, g_sem_a)

            scale(cb, rows_b)
            scatter(cb, rows_b, w_sem_b)
            return carry2

        lax.fori_loop(0, PAIRS, pair, 0)
        return carry

    lax.fori_loop(0, NSLAB, slab, 0)
    drain(rows_a, w_sem_a)
    drain(rows_b, w_sem_b)
    plsc.subcore_barrier()

    bvs = [b_v[0, pl.ds(pl.multiple_of(col0 + k2 * 16, 16), 16)]
           for k2 in range(HALF // 16)]

    def fin_blk(k, carry):
        sl = pl.ds(r0 + k * CHUNK, CHUNK)
        pltpu.sync_copy(accS.at[sl], rows_a)
        pltpu.sync_copy(denS.at[sl], d80)
        for q in range(CHUNK // 16):
            d16 = d80[pl.ds(q * 16, 16)]
            inv = 1.0 / jnp.where(d16 > 0.0, d16, 1.0)
            for u in range(16):
                iv = inv[u]
                ei = q * 16 + u
                for k2 in range(HALF // 16):
                    sl2 = pl.ds(k2 * 16, 16)
                    rows_a[ei, sl2] = jnp.maximum(
                        rows_a[ei, sl2] * iv + bvs[k2], 0.0)
        pltpu.sync_copy(rows_a, out_hbm.at[sl, pl.ds(col0, HALF)])
        return carry

    lax.fori_loop(0, n_my, fin_blk, 0)


_sc_edge = pl.kernel(
    _sc_edge_body,
    out_type=jax.ShapeDtypeStruct((N, F_OUT), jnp.float32),
    mesh=plsc.VectorSubcoreMesh(core_axis_name="c", subcore_axis_name="s"),
    compiler_params=pltpu.CompilerParams(needs_layout_passes=False,
                                         use_tc_tiling_on_sc=False),
    scratch_types=[
        pltpu.VMEM_SHARED((N, HALF), jnp.float32),
        pltpu.VMEM_SHARED((N, HALF), jnp.float32),
        pltpu.VMEM_SHARED((N,), jnp.float32),
        pltpu.VMEM((N,), jnp.float32),
        pltpu.VMEM((N,), jnp.float32),
        pltpu.VMEM((1, 16), jnp.float32),
        pltpu.VMEM((1, F_OUT), jnp.float32),
        pltpu.VMEM((CHUNK,), jnp.float32),
        pltpu.VMEM((SLAB, CHUNK), jnp.int32),
        pltpu.VMEM((SLAB, CHUNK), jnp.int32),
        pltpu.VMEM((SLAB, CHUNK), jnp.float32),
        pltpu.VMEM((SLAB, CHUNK), jnp.float32),
        pltpu.VMEM((CHUNK, HALF), jnp.float32),
        pltpu.VMEM((CHUNK, HALF), jnp.float32),
        pltpu.VMEM((2000,), jnp.float32),
        pltpu.SemaphoreType.DMA,
        pltpu.SemaphoreType.DMA,
        pltpu.SemaphoreType.DMA,
        pltpu.SemaphoreType.DMA,
    ],
)


def kernel(x, edge_index, adj_values, W, b, a_self, a_neigh):
    row = edge_index[0].astype(jnp.int32).reshape(E // CHUNK, CHUNK)
    col = edge_index[1].astype(jnp.int32).reshape(E // CHUNK, CHUNK)
    a2 = jnp.concatenate([a_self, a_neigh], axis=1)
    feat, s2, n2, m16 = _tc_pre(x, W, a2)
    adj = adj_values.astype(jnp.float32).reshape(E // CHUNK, CHUNK)
    out = _sc_edge(feat, s2.reshape(N), n2.reshape(N), m16, row, col, adj,
                   b.reshape(1, F_OUT))
    return out


# speedup vs baseline: 1.1085x; 1.1085x over previous
import jax
import jax.numpy as jnp
from jax import lax
from jax.experimental import pallas as pl
from jax.experimental.pallas import tpu as pltpu
from jax.experimental.pallas import tpu_sc as plsc

N = 10000
E = 320000
F_IN = 128
F_OUT = 128
HALF = F_OUT // 2
NS = 16
NC = 2
ROWS_PER_TILE = 640
ROWS_LAST = N - 15 * ROWS_PER_TILE
EDGES_PER_TILE = E // NS
CHUNK = 80
NCHUNK = EDGES_PER_TILE // CHUNK
SLAB = 10
NSLAB = NCHUNK // SLAB


def _leaky(v):
    return jnp.where(v >= 0.0, v, 0.2 * v)


def _tc_pre_body(x_ref, w_ref, a_ref, feat_ref, s_ref, n_ref, m_ref):
    f = jnp.dot(x_ref[...], w_ref[...],
                preferred_element_type=jnp.float32,
                precision=lax.Precision.HIGHEST)
    feat_ref[...] = f
    attn_t = lax.dot_general(a_ref[...], f, (((0,), (1,)), ((), ())),
                             preferred_element_type=jnp.float32,
                             precision=lax.Precision.HIGHEST)
    s_ref[...] = attn_t[0:1, :]
    n_ref[...] = attn_t[1:2, :]
    m_ref[...] = jnp.broadcast_to(jnp.max(attn_t[1, :]), (1, 16))


_tc_pre = pl.pallas_call(
    _tc_pre_body,
    out_shape=[
        jax.ShapeDtypeStruct((N, F_OUT), jnp.float32),
        jax.ShapeDtypeStruct((1, N), jnp.float32),
        jax.ShapeDtypeStruct((1, N), jnp.float32),
        jax.ShapeDtypeStruct((1, 16), jnp.float32),
    ],
)


def _sc_edge_body(feat_hbm, s_hbm, n_hbm, m_hbm, row_hbm, col_hbm, adj_hbm,
                  b_hbm, out_hbm,
                  table, accS, denS,
                  s_v, n_v, m_v, b_v, d80, row_b, col_b, adj_b,
                  p_a, p_b, rows_a, rows_b, z1_v, g_sem_a, g_sem_b):
    c = lax.axis_index("c")
    t = lax.axis_index("s")
    r0 = pl.multiple_of(t * ROWS_PER_TILE, 8)
    col0 = pl.multiple_of(c * HALF, 16)
    n_stage = ROWS_PER_TILE // CHUNK
    n_my = jnp.where(t < 15, n_stage, ROWS_LAST // CHUNK)

    z16 = jnp.zeros((16,), jnp.float32)
    for i in range(CHUNK):
        for k in range(HALF // 16):
            rows_a[i, pl.ds(k * 16, 16)] = z16

    def zero_blk(k, carry):
        pltpu.sync_copy(rows_a, accS.at[pl.ds(r0 + k * CHUNK, CHUNK)])
        return carry

    lax.fori_loop(0, n_my, zero_blk, 0)

    def stage_blk(k, carry):
        sl = pl.ds(r0 + k * CHUNK, CHUNK)
        pltpu.sync_copy(feat_hbm.at[sl, pl.ds(col0, HALF)], rows_b)
        pltpu.sync_copy(rows_b, table.at[sl])
        return carry

    lax.fori_loop(0, n_my, stage_blk, 0)

    @pl.when(t < 5)
    def _():
        for i in range(2000 // 16):
            z1_v[pl.ds(i * 16, 16)] = z16
        pltpu.sync_copy(z1_v, denS.at[pl.ds(t * 2000, 2000)])

    pltpu.sync_copy(s_hbm, s_v)
    pltpu.sync_copy(n_hbm, n_v)
    pltpu.sync_copy(m_hbm, m_v)
    pltpu.sync_copy(b_hbm, b_v)
    plsc.subcore_barrier()

    m = m_v[0, pl.ds(0, 16)][0]
    z16i = jnp.zeros((16,), jnp.int32)
    c0 = t * NCHUNK

    def compute_p(ci, p_ref):
        for j in range(CHUNK // 16):
            sl = pl.ds(j * 16, 16)
            a_s = plsc.load_gather(s_v, [row_b[ci, sl]])
            a_n = plsc.load_gather(n_v, [col_b[ci, sl]])
            e = _leaky(a_s + a_n) * adj_b[ci, sl]
            p_ref[sl] = jnp.exp(e - _leaky(a_s + m))

    def scale(rows_ref, p_ref):
        for j in range(CHUNK // 16):
            p16 = p_ref[pl.ds(j * 16, 16)]
            for u in range(16):
                pe = p16[u]
                ei = j * 16 + u
                for k2 in range(HALF // 16):
                    sl2 = pl.ds(k2 * 16, 16)
                    rows_ref[ei, sl2] = rows_ref[ei, sl2] * pe

    def scatter(ci, rows_ref, p_ref):
        pltpu.sync_copy(rows_ref, accS.at[row_b.at[ci]], add=True)
        pltpu.sync_copy(p_ref, denS.at[row_b.at[ci]], add=True)

    PAIRS = SLAB // 2

    def slab(sb, carry):
        sl0 = pl.multiple_of(c0 + sb * SLAB, 2)
        pltpu.sync_copy(row_hbm.at[pl.ds(sl0, SLAB)], row_b)
        pltpu.sync_copy(col_hbm.at[pl.ds(sl0, SLAB)], col_b)
        pltpu.sync_copy(adj_hbm.at[pl.ds(sl0, SLAB)], adj_b)
        pltpu.async_copy(table.at[col_b.at[0]], rows_a, g_sem_a)

        def pair(i, carry2):
            ca = 2 * i
            cb = 2 * i + 1
            compute_p(ca, p_a)
            pltpu.make_async_copy(table.at[col_b.at[ca]], rows_a,
                                  g_sem_a).wait()
            pltpu.async_copy(table.at[col_b.at[cb]], rows_b, g_sem_b)
            scale(rows_a, p_a)
            scatter(ca, rows_a, p_a)
            compute_p(cb, p_b)
            pltpu.make_async_copy(table.at[col_b.at[cb]], rows_b,
                                  g_sem_b).wait()

            @pl.when(i < PAIRS - 1)
            def _():
                pltpu.async_copy(table.at[col_b.at[cb + 1]], rows_a, g_sem_a)

            scale(rows_b, p_b)
            scatter(cb, rows_b, p_b)
            return carry2

        lax.fori_loop(0, PAIRS, pair, 0)
        return carry

    lax.fori_loop(0, NSLAB, slab, 0)
    plsc.subcore_barrier()

    bvs = [b_v[0, pl.ds(pl.multiple_of(col0 + k2 * 16, 16), 16)]
           for k2 in range(HALF // 16)]

    def fin_blk(k, carry):
        sl = pl.ds(r0 + k * CHUNK, CHUNK)
        pltpu.sync_copy(accS.at[sl], rows_a)
        pltpu.sync_copy(denS.at[sl], d80)
        for q in range(CHUNK // 16):
            d16 = d80[pl.ds(q * 16, 16)]
            inv = 1.0 / jnp.where(d16 > 0.0, d16, 1.0)
            for u in range(16):
                iv = inv[u]
                ei = q * 16 + u
                for k2 in range(HALF // 16):
                    sl2 = pl.ds(k2 * 16, 16)
                    rows_a[ei, sl2] = jnp.maximum(
                        rows_a[ei, sl2] * iv + bvs[k2], 0.0)
        pltpu.sync_copy(rows_a, out_hbm.at[sl, pl.ds(col0, HALF)])
        return carry

    lax.fori_loop(0, n_my, fin_blk, 0)


_sc_edge = pl.kernel(
    _sc_edge_body,
    out_type=jax.ShapeDtypeStruct((N, F_OUT), jnp.float32),
    mesh=plsc.VectorSubcoreMesh(core_axis_name="c", subcore_axis_name="s"),
    compiler_params=pltpu.CompilerParams(needs_layout_passes=False,
                                         use_tc_tiling_on_sc=False),
    scratch_types=[
        pltpu.VMEM_SHARED((N, HALF), jnp.float32),
        pltpu.VMEM_SHARED((N, HALF), jnp.float32),
        pltpu.VMEM_SHARED((N,), jnp.float32),
        pltpu.VMEM((N,), jnp.float32),
        pltpu.VMEM((N,), jnp.float32),
        pltpu.VMEM((1, 16), jnp.float32),
        pltpu.VMEM((1, F_OUT), jnp.float32),
        pltpu.VMEM((CHUNK,), jnp.float32),
        pltpu.VMEM((SLAB, CHUNK), jnp.int32),
        pltpu.VMEM((SLAB, CHUNK), jnp.int32),
        pltpu.VMEM((SLAB, CHUNK), jnp.float32),
        pltpu.VMEM((CHUNK,), jnp.float32),
        pltpu.VMEM((CHUNK,), jnp.float32),
        pltpu.VMEM((CHUNK, HALF), jnp.float32),
        pltpu.VMEM((CHUNK, HALF), jnp.float32),
        pltpu.VMEM((2000,), jnp.float32),
        pltpu.SemaphoreType.DMA,
        pltpu.SemaphoreType.DMA,
    ],
)


def kernel(x, edge_index, adj_values, W, b, a_self, a_neigh):
    row = edge_index[0].astype(jnp.int32).reshape(E // CHUNK, CHUNK)
    col = edge_index[1].astype(jnp.int32).reshape(E // CHUNK, CHUNK)
    a2 = jnp.concatenate([a_self, a_neigh], axis=1)
    feat, s2, n2, m16 = _tc_pre(x, W, a2)
    adj = adj_values.astype(jnp.float32).reshape(E // CHUNK, CHUNK)
    out = _sc_edge(feat, s2.reshape(N), n2.reshape(N), m16, row, col, adj,
                   b.reshape(1, F_OUT))
    return out


# speedup vs baseline: 1.2319x; 1.1113x over previous
import jax
import jax.numpy as jnp
from jax import lax
from jax.experimental import pallas as pl
from jax.experimental.pallas import tpu as pltpu
from jax.experimental.pallas import tpu_sc as plsc

N = 10000
E = 320000
F_IN = 128
F_OUT = 128
HALF = F_OUT // 2
NS = 16
NC = 2
ROWS_PER_TILE = 640
ROWS_LAST = N - 15 * ROWS_PER_TILE
EDGES_PER_TILE = E // NS
CHUNK = 80
NCHUNK = EDGES_PER_TILE // CHUNK
SLAB = 20
NSLAB = NCHUNK // SLAB


def _leaky(v):
    return jnp.where(v >= 0.0, v, 0.2 * v)


def _tc_pre_body(x_ref, w_ref, a_ref, feat_ref, s_ref, n_ref, m_ref):
    f = jnp.dot(x_ref[...], w_ref[...],
                preferred_element_type=jnp.float32,
                precision=lax.Precision.HIGHEST)
    feat_ref[...] = f
    attn_t = lax.dot_general(a_ref[...], f, (((0,), (1,)), ((), ())),
                             preferred_element_type=jnp.float32,
                             precision=lax.Precision.HIGHEST)
    s_ref[...] = attn_t[0:1, :]
    n_ref[...] = attn_t[1:2, :]
    m_ref[...] = jnp.broadcast_to(jnp.max(attn_t[1, :]), (1, 16))


_tc_pre = pl.pallas_call(
    _tc_pre_body,
    out_shape=[
        jax.ShapeDtypeStruct((N, F_OUT), jnp.float32),
        jax.ShapeDtypeStruct((1, N), jnp.float32),
        jax.ShapeDtypeStruct((1, N), jnp.float32),
        jax.ShapeDtypeStruct((1, 16), jnp.float32),
    ],
)


def _sc_edge_body(feat_hbm, s_hbm, n_hbm, m_hbm, row_hbm, col_hbm, adj_hbm,
                  b_hbm, out_hbm,
                  table, accS, denS,
                  s_v, n_v, m_v, b_v, d80, row_b, col_b, adj_b,
                  p_a, p_b, rows_a, rows_b, z1_v, g_sem_a, g_sem_b):
    c = lax.axis_index("c")
    t = lax.axis_index("s")
    r0 = pl.multiple_of(t * ROWS_PER_TILE, 8)
    col0 = pl.multiple_of(c * HALF, 16)
    n_stage = ROWS_PER_TILE // CHUNK
    n_my = jnp.where(t < 15, n_stage, ROWS_LAST // CHUNK)

    z16 = jnp.zeros((16,), jnp.float32)
    for i in range(CHUNK):
        for k in range(HALF // 16):
            rows_a[i, pl.ds(k * 16, 16)] = z16

    def zero_blk(k, carry):
        pltpu.sync_copy(rows_a, accS.at[pl.ds(r0 + k * CHUNK, CHUNK)])
        return carry

    lax.fori_loop(0, n_my, zero_blk, 0)

    def stage_blk(k, carry):
        sl = pl.ds(r0 + k * CHUNK, CHUNK)
        pltpu.sync_copy(feat_hbm.at[sl, pl.ds(col0, HALF)], rows_b)
        pltpu.sync_copy(rows_b, table.at[sl])
        return carry

    lax.fori_loop(0, n_my, stage_blk, 0)

    @pl.when(t < 5)
    def _():
        for i in range(2000 // 16):
            z1_v[pl.ds(i * 16, 16)] = z16
        pltpu.sync_copy(z1_v, denS.at[pl.ds(t * 2000, 2000)])

    pltpu.sync_copy(s_hbm, s_v)
    pltpu.sync_copy(n_hbm, n_v)
    pltpu.sync_copy(m_hbm, m_v)
    pltpu.sync_copy(b_hbm, b_v)
    plsc.subcore_barrier()

    m = m_v[0, pl.ds(0, 16)][0]
    z16i = jnp.zeros((16,), jnp.int32)
    c0 = t * NCHUNK

    def compute_p(ci, p_ref):
        for j in range(CHUNK // 16):
            sl = pl.ds(j * 16, 16)
            a_s = plsc.load_gather(s_v, [row_b[ci, sl]])
            a_n = plsc.load_gather(n_v, [col_b[ci, sl]])
            e = _leaky(a_s + a_n) * adj_b[ci, sl]
            p_ref[sl] = jnp.exp(e - _leaky(a_s + m))

    def scale(rows_ref, p_ref):
        for j in range(CHUNK // 16):
            p16 = p_ref[pl.ds(j * 16, 16)]
            for u in range(16):
                pe = p16[u]
                ei = j * 16 + u
                for k2 in range(HALF // 16):
                    sl2 = pl.ds(k2 * 16, 16)
                    rows_ref[ei, sl2] = rows_ref[ei, sl2] * pe

    def scatter(ci, rows_ref, p_ref):
        pltpu.sync_copy(rows_ref, accS.at[row_b.at[ci]], add=True)
        pltpu.sync_copy(p_ref, denS.at[row_b.at[ci]], add=True)

    PAIRS = SLAB // 2

    def slab(sb, carry):
        sl0 = pl.multiple_of(c0 + sb * SLAB, 2)
        pltpu.sync_copy(row_hbm.at[pl.ds(sl0, SLAB)], row_b)
        pltpu.sync_copy(col_hbm.at[pl.ds(sl0, SLAB)], col_b)
        pltpu.sync_copy(adj_hbm.at[pl.ds(sl0, SLAB)], adj_b)
        pltpu.async_copy(table.at[col_b.at[0]], rows_a, g_sem_a)

        def pair(i, carry2):
            ca = 2 * i
            cb = 2 * i + 1
            compute_p(ca, p_a)
            pltpu.make_async_copy(table.at[col_b.at[ca]], rows_a,
                                  g_sem_a).wait()
            pltpu.async_copy(table.at[col_b.at[cb]], rows_b, g_sem_b)
            scale(rows_a, p_a)
            scatter(ca, rows_a, p_a)
            compute_p(cb, p_b)
            pltpu.make_async_copy(table.at[col_b.at[cb]], rows_b,
                                  g_sem_b).wait()

            @pl.when(i < PAIRS - 1)
            def _():
                pltpu.async_copy(table.at[col_b.at[cb + 1]], rows_a, g_sem_a)

            scale(rows_b, p_b)
            scatter(cb, rows_b, p_b)
            return carry2

        lax.fori_loop(0, PAIRS, pair, 0)
        return carry

    lax.fori_loop(0, NSLAB, slab, 0)
    plsc.subcore_barrier()

    bvs = [b_v[0, pl.ds(pl.multiple_of(col0 + k2 * 16, 16), 16)]
           for k2 in range(HALF // 16)]

    def fin_blk(k, carry):
        sl = pl.ds(r0 + k * CHUNK, CHUNK)
        pltpu.sync_copy(accS.at[sl], rows_a)
        pltpu.sync_copy(denS.at[sl], d80)
        for q in range(CHUNK // 16):
            d16 = d80[pl.ds(q * 16, 16)]
            inv = 1.0 / jnp.where(d16 > 0.0, d16, 1.0)
            for u in range(16):
                iv = inv[u]
                ei = q * 16 + u
                for k2 in range(HALF // 16):
                    sl2 = pl.ds(k2 * 16, 16)
                    rows_a[ei, sl2] = jnp.maximum(
                        rows_a[ei, sl2] * iv + bvs[k2], 0.0)
        pltpu.sync_copy(rows_a, out_hbm.at[sl, pl.ds(col0, HALF)])
        return carry

    lax.fori_loop(0, n_my, fin_blk, 0)


_sc_edge = pl.kernel(
    _sc_edge_body,
    out_type=jax.ShapeDtypeStruct((N, F_OUT), jnp.float32),
    mesh=plsc.VectorSubcoreMesh(core_axis_name="c", subcore_axis_name="s"),
    compiler_params=pltpu.CompilerParams(needs_layout_passes=False,
                                         use_tc_tiling_on_sc=False),
    scratch_types=[
        pltpu.VMEM_SHARED((N, HALF), jnp.float32),
        pltpu.VMEM_SHARED((N, HALF), jnp.float32),
        pltpu.VMEM_SHARED((N,), jnp.float32),
        pltpu.VMEM((N,), jnp.float32),
        pltpu.VMEM((N,), jnp.float32),
        pltpu.VMEM((1, 16), jnp.float32),
        pltpu.VMEM((1, F_OUT), jnp.float32),
        pltpu.VMEM((CHUNK,), jnp.float32),
        pltpu.VMEM((SLAB, CHUNK), jnp.int32),
        pltpu.VMEM((SLAB, CHUNK), jnp.int32),
        pltpu.VMEM((SLAB, CHUNK), jnp.float32),
        pltpu.VMEM((CHUNK,), jnp.float32),
        pltpu.VMEM((CHUNK,), jnp.float32),
        pltpu.VMEM((CHUNK, HALF), jnp.float32),
        pltpu.VMEM((CHUNK, HALF), jnp.float32),
        pltpu.VMEM((2000,), jnp.float32),
        pltpu.SemaphoreType.DMA,
        pltpu.SemaphoreType.DMA,
    ],
)


def kernel(x, edge_index, adj_values, W, b, a_self, a_neigh):
    row = edge_index[0].astype(jnp.int32).reshape(E // CHUNK, CHUNK)
    col = edge_index[1].astype(jnp.int32).reshape(E // CHUNK, CHUNK)
    a2 = jnp.concatenate([a_self, a_neigh], axis=1)
    feat, s2, n2, m16 = _tc_pre(x, W, a2)
    adj = adj_values.astype(jnp.float32).reshape(E // CHUNK, CHUNK)
    out = _sc_edge(feat, s2.reshape(N), n2.reshape(N), m16, row, col, adj,
                   b.reshape(1, F_OUT))
    return out


# speedup vs baseline: 1.2685x; 1.0297x over previous
import jax
import jax.numpy as jnp
from jax import lax
from jax.experimental import pallas as pl
from jax.experimental.pallas import tpu as pltpu
from jax.experimental.pallas import tpu_sc as plsc

N = 10000
E = 320000
F_IN = 128
F_OUT = 128
HALF = F_OUT // 2
NS = 16
NC = 2
ROWS_PER_TILE = 640
ROWS_LAST = N - 15 * ROWS_PER_TILE
EDGES_PER_TILE = E // NS
CHUNK = 80
NCHUNK = EDGES_PER_TILE // CHUNK
SLAB = 50
NSLAB = NCHUNK // SLAB


def _leaky(v):
    return jnp.where(v >= 0.0, v, 0.2 * v)


def _tc_pre_body(x_ref, w_ref, a_ref, feat_ref, s_ref, n_ref, m_ref):
    f = jnp.dot(x_ref[...], w_ref[...],
                preferred_element_type=jnp.float32,
                precision=lax.Precision.HIGHEST)
    feat_ref[...] = f
    attn_t = lax.dot_general(a_ref[...], f, (((0,), (1,)), ((), ())),
                             preferred_element_type=jnp.float32,
                             precision=lax.Precision.HIGHEST)
    s_ref[...] = attn_t[0:1, :]
    n_ref[...] = attn_t[1:2, :]
    m_ref[...] = jnp.broadcast_to(jnp.max(attn_t[1, :]), (1, 16))


_tc_pre = pl.pallas_call(
    _tc_pre_body,
    out_shape=[
        jax.ShapeDtypeStruct((N, F_OUT), jnp.float32),
        jax.ShapeDtypeStruct((1, N), jnp.float32),
        jax.ShapeDtypeStruct((1, N), jnp.float32),
        jax.ShapeDtypeStruct((1, 16), jnp.float32),
    ],
)


def _sc_edge_body(feat_hbm, s_hbm, n_hbm, m_hbm, row_hbm, col_hbm, adj_hbm,
                  b_hbm, out_hbm,
                  table, accS, denS,
                  s_v, n_v, m_v, b_v, d80, row_b, col_b, adj_b,
                  p_a, p_b, rows_a, rows_b, z1_v, g_sem_a, g_sem_b):
    c = lax.axis_index("c")
    t = lax.axis_index("s")
    r0 = pl.multiple_of(t * ROWS_PER_TILE, 8)
    col0 = pl.multiple_of(c * HALF, 16)
    n_stage = ROWS_PER_TILE // CHUNK
    n_my = jnp.where(t < 15, n_stage, ROWS_LAST // CHUNK)

    z16 = jnp.zeros((16,), jnp.float32)
    for i in range(CHUNK):
        for k in range(HALF // 16):
            rows_a[i, pl.ds(k * 16, 16)] = z16

    def zero_blk(k, carry):
        pltpu.sync_copy(rows_a, accS.at[pl.ds(r0 + k * CHUNK, CHUNK)])
        return carry

    lax.fori_loop(0, n_my, zero_blk, 0)

    def stage_blk(k, carry):
        sl = pl.ds(r0 + k * CHUNK, CHUNK)
        pltpu.sync_copy(feat_hbm.at[sl, pl.ds(col0, HALF)], rows_b)
        pltpu.sync_copy(rows_b, table.at[sl])
        return carry

    lax.fori_loop(0, n_my, stage_blk, 0)

    @pl.when(t < 5)
    def _():
        for i in range(2000 // 16):
            z1_v[pl.ds(i * 16, 16)] = z16
        pltpu.sync_copy(z1_v, denS.at[pl.ds(t * 2000, 2000)])

    pltpu.sync_copy(s_hbm, s_v)
    pltpu.sync_copy(n_hbm, n_v)
    pltpu.sync_copy(m_hbm, m_v)
    pltpu.sync_copy(b_hbm, b_v)
    plsc.subcore_barrier()

    m = m_v[0, pl.ds(0, 16)][0]
    z16i = jnp.zeros((16,), jnp.int32)
    c0 = t * NCHUNK

    def compute_p(ci, p_ref):
        for j in range(CHUNK // 16):
            sl = pl.ds(j * 16, 16)
            a_s = plsc.load_gather(s_v, [row_b[ci, sl]])
            a_n = plsc.load_gather(n_v, [col_b[ci, sl]])
            e = _leaky(a_s + a_n) * adj_b[ci, sl]
            p_ref[sl] = jnp.exp(e - _leaky(a_s + m))

    def scale(rows_ref, p_ref):
        for j in range(CHUNK // 16):
            p16 = p_ref[pl.ds(j * 16, 16)]
            for u in range(16):
                pe = p16[u]
                ei = j * 16 + u
                for k2 in range(HALF // 16):
                    sl2 = pl.ds(k2 * 16, 16)
                    rows_ref[ei, sl2] = rows_ref[ei, sl2] * pe

    def scatter(ci, rows_ref, p_ref):
        pltpu.sync_copy(rows_ref, accS.at[row_b.at[ci]], add=True)
        pltpu.sync_copy(p_ref, denS.at[row_b.at[ci]], add=True)

    PAIRS = SLAB // 2

    def slab(sb, carry):
        sl0 = pl.multiple_of(c0 + sb * SLAB, 2)
        pltpu.sync_copy(row_hbm.at[pl.ds(sl0, SLAB)], row_b)
        pltpu.sync_copy(col_hbm.at[pl.ds(sl0, SLAB)], col_b)
        pltpu.sync_copy(adj_hbm.at[pl.ds(sl0, SLAB)], adj_b)
        pltpu.async_copy(table.at[col_b.at[0]], rows_a, g_sem_a)

        def pair(i, carry2):
            ca = 2 * i
            cb = 2 * i + 1
            compute_p(ca, p_a)
            pltpu.make_async_copy(table.at[col_b.at[ca]], rows_a,
                                  g_sem_a).wait()
            pltpu.async_copy(table.at[col_b.at[cb]], rows_b, g_sem_b)
            scale(rows_a, p_a)
            scatter(ca, rows_a, p_a)
            compute_p(cb, p_b)
            pltpu.make_async_copy(table.at[col_b.at[cb]], rows_b,
                                  g_sem_b).wait()

            @pl.when(i < PAIRS - 1)
            def _():
                pltpu.async_copy(table.at[col_b.at[cb + 1]], rows_a, g_sem_a)

            scale(rows_b, p_b)
            scatter(cb, rows_b, p_b)
            return carry2

        lax.fori_loop(0, PAIRS, pair, 0)
        return carry

    lax.fori_loop(0, NSLAB, slab, 0)
    plsc.subcore_barrier()

    bvs = [b_v[0, pl.ds(pl.multiple_of(col0 + k2 * 16, 16), 16)]
           for k2 in range(HALF // 16)]

    def fin_blk(k, carry):
        sl = pl.ds(r0 + k * CHUNK, CHUNK)
        pltpu.sync_copy(accS.at[sl], rows_a)
        pltpu.sync_copy(denS.at[sl], d80)
        for q in range(CHUNK // 16):
            d16 = d80[pl.ds(q * 16, 16)]
            inv = 1.0 / jnp.where(d16 > 0.0, d16, 1.0)
            for u in range(16):
                iv = inv[u]
                ei = q * 16 + u
                for k2 in range(HALF // 16):
                    sl2 = pl.ds(k2 * 16, 16)
                    rows_a[ei, sl2] = jnp.maximum(
                        rows_a[ei, sl2] * iv + bvs[k2], 0.0)
        pltpu.sync_copy(rows_a, out_hbm.at[sl, pl.ds(col0, HALF)])
        return carry

    lax.fori_loop(0, n_my, fin_blk, 0)


_sc_edge = pl.kernel(
    _sc_edge_body,
    out_type=jax.ShapeDtypeStruct((N, F_OUT), jnp.float32),
    mesh=plsc.VectorSubcoreMesh(core_axis_name="c", subcore_axis_name="s"),
    compiler_params=pltpu.CompilerParams(needs_layout_passes=False,
                                         use_tc_tiling_on_sc=False),
    scratch_types=[
        pltpu.VMEM_SHARED((N, HALF), jnp.float32),
        pltpu.VMEM_SHARED((N, HALF), jnp.float32),
        pltpu.VMEM_SHARED((N,), jnp.float32),
        pltpu.VMEM((N,), jnp.float32),
        pltpu.VMEM((N,), jnp.float32),
        pltpu.VMEM((1, 16), jnp.float32),
        pltpu.VMEM((1, F_OUT), jnp.float32),
        pltpu.VMEM((CHUNK,), jnp.float32),
        pltpu.VMEM((SLAB, CHUNK), jnp.int32),
        pltpu.VMEM((SLAB, CHUNK), jnp.int32),
        pltpu.VMEM((SLAB, CHUNK), jnp.float32),
        pltpu.VMEM((CHUNK,), jnp.float32),
        pltpu.VMEM((CHUNK,), jnp.float32),
        pltpu.VMEM((CHUNK, HALF), jnp.float32),
        pltpu.VMEM((CHUNK, HALF), jnp.float32),
        pltpu.VMEM((2000,), jnp.float32),
        pltpu.SemaphoreType.DMA,
        pltpu.SemaphoreType.DMA,
    ],
)


def kernel(x, edge_index, adj_values, W, b, a_self, a_neigh):
    row = edge_index[0].astype(jnp.int32).reshape(E // CHUNK, CHUNK)
    col = edge_index[1].astype(jnp.int32).reshape(E // CHUNK, CHUNK)
    a2 = jnp.concatenate([a_self, a_neigh], axis=1)
    feat, s2, n2, m16 = _tc_pre(x, W, a2)
    adj = adj_values.astype(jnp.float32).reshape(E // CHUNK, CHUNK)
    out = _sc_edge(feat, s2.reshape(N), n2.reshape(N), m16, row, col, adj,
                   b.reshape(1, F_OUT))
    return out
